# baseline (speedup 1.0000x reference)
import functools

import jax
import jax.numpy as jnp
from jax import lax
from jax.experimental import pallas as pl
from jax.experimental.pallas import tpu as pltpu
from jax.experimental.pallas import tpu_sc as plsc


def _sc_gather(table, idx, b_per_w, nc):
    B = idx.shape[0]
    D = table.shape[1]
    mesh = plsc.VectorSubcoreMesh(core_axis_name="c", subcore_axis_name="s")

    @functools.partial(
        pl.kernel,
        mesh=mesh,
        out_type=jax.ShapeDtypeStruct((B, D), jnp.float32),
        scratch_types=[
            pltpu.VMEM((b_per_w,), jnp.int32),
            pltpu.VMEM((b_per_w, D), jnp.float32),
            pltpu.SemaphoreType.DMA,
        ],
        compiler_params=pltpu.CompilerParams(use_tc_tiling_on_sc=False),
    )
    def gather_kernel(table_hbm, idx_hbm, out_hbm, idx_v, rows_v, sem):
        wid = lax.axis_index("s") * nc + lax.axis_index("c")
        base = wid * b_per_w
        pltpu.sync_copy(idx_hbm.at[pl.ds(base, b_per_w)], idx_v)
        pltpu.async_copy(table_hbm.at[idx_v], rows_v, sem).wait()
        pltpu.sync_copy(rows_v, out_hbm.at[pl.ds(base, b_per_w)])

    return gather_kernel(table, idx)


def _tc_matmul_bias(c2, wt, bias2, bm):
    M, K = c2.shape
    N = wt.shape[0]

    def body(a_ref, wt_ref, b_ref, o_ref):
        acc = lax.dot_general(
            a_ref[...], wt_ref[...],
            dimension_numbers=(((1,), (1,)), ((), ())),
            preferred_element_type=jnp.float32,
        )
        o_ref[...] = acc + b_ref[...]

    return pl.pallas_call(
        body,
        grid=(M // bm,),
        in_specs=[
            pl.BlockSpec((bm, K), lambda i: (i, 0)),
            pl.BlockSpec((N, K), lambda i: (0, 0)),
            pl.BlockSpec((1, N), lambda i: (0, 0)),
        ],
        out_specs=pl.BlockSpec((bm, N), lambda i: (i, 0)),
        out_shape=jax.ShapeDtypeStruct((M, N), jnp.float32),
        compiler_params=pltpu.CompilerParams(
            dimension_semantics=("parallel",),
        ),
    )(c2, wt, bias2)


def kernel(cellgene_embedding, genes_oi, weight1, bias1):
    M, G, NI = cellgene_embedding.shape
    NO = bias1.shape[0]

    info = plsc.get_sparse_core_info()
    nw = info.num_cores * info.num_subcores
    chunk = 8 * nw
    Bp = ((G + chunk - 1) // chunk) * chunk
    idx = jnp.pad(genes_oi.astype(jnp.int32), (0, Bp - G))

    gathered = _sc_gather(weight1, idx, Bp // nw, info.num_cores)
    wt = gathered[:G].reshape(G * NI, NO).T
    c2 = cellgene_embedding.reshape(M, G * NI)
    bias2 = bias1.reshape(1, NO)

    return _tc_matmul_bias(c2, wt, bias2, bm=128)


# speedup vs baseline: 1.7043x; 1.7043x over previous
import functools

import jax
import jax.numpy as jnp
from jax import lax
from jax.experimental import pallas as pl
from jax.experimental.pallas import tpu as pltpu
from jax.experimental.pallas import tpu_sc as plsc


def _sc_gather(table, idx, b_per_w, nc):
    B = idx.shape[0]
    D = table.shape[1]
    mesh = plsc.VectorSubcoreMesh(core_axis_name="c", subcore_axis_name="s")

    @functools.partial(
        pl.kernel,
        mesh=mesh,
        out_type=jax.ShapeDtypeStruct((B, D), jnp.float32),
        scratch_types=[
            pltpu.VMEM((b_per_w,), jnp.int32),
            pltpu.SMEM((b_per_w,), jnp.int32),
            pltpu.VMEM((b_per_w, D), jnp.float32),
            pltpu.SemaphoreType.DMA,
        ],
    )
    def gather_kernel(table_hbm, idx_hbm, out_hbm, idx_v, idx_s, rows_v, sem):
        wid = lax.axis_index("s") * nc + lax.axis_index("c")
        base = wid * b_per_w
        pltpu.sync_copy(idx_hbm.at[pl.ds(base, b_per_w)], idx_v)

        def issue_chunk(j, carry):
            vec = idx_v[pl.ds(j * 16, 16)]
            for lane in range(16):
                g = vec[lane]
                pltpu.make_async_copy(
                    table_hbm.at[pl.ds(g, 1)],
                    rows_v.at[pl.ds(j * 16 + lane, 1)],
                    sem,
                ).start()
            return carry

        lax.fori_loop(0, b_per_w // 16, issue_chunk, 0)
        pltpu.make_async_copy(table_hbm.at[pl.ds(0, b_per_w)], rows_v, sem).wait()
        pltpu.sync_copy(rows_v, out_hbm.at[pl.ds(base, b_per_w)])

    return gather_kernel(table, idx)


def _tc_matmul_bias(c2, wt, bias2, bm):
    M, K = c2.shape
    N = wt.shape[0]

    def body(a_ref, wt_ref, b_ref, o_ref):
        acc = lax.dot_general(
            a_ref[...], wt_ref[...],
            dimension_numbers=(((1,), (1,)), ((), ())),
            preferred_element_type=jnp.float32,
        )
        o_ref[...] = acc + b_ref[...]

    return pl.pallas_call(
        body,
        grid=(M // bm,),
        in_specs=[
            pl.BlockSpec((bm, K), lambda i: (i, 0)),
            pl.BlockSpec((N, K), lambda i: (0, 0)),
            pl.BlockSpec((1, N), lambda i: (0, 0)),
        ],
        out_specs=pl.BlockSpec((bm, N), lambda i: (i, 0)),
        out_shape=jax.ShapeDtypeStruct((M, N), jnp.float32),
        compiler_params=pltpu.CompilerParams(
            dimension_semantics=("parallel",),
        ),
    )(c2, wt, bias2)


def kernel(cellgene_embedding, genes_oi, weight1, bias1):
    M, G, NI = cellgene_embedding.shape
    NO = bias1.shape[0]

    info = plsc.get_sparse_core_info()
    nw = info.num_cores * info.num_subcores
    chunk = 8 * nw
    Bp = ((G + chunk - 1) // chunk) * chunk
    idx = jnp.pad(genes_oi.astype(jnp.int32), (0, Bp - G))

    gathered = _sc_gather(weight1, idx, Bp // nw, info.num_cores)
    wt = gathered[:G].reshape(G * NI, NO).T
    c2 = cellgene_embedding.reshape(M, G * NI)
    bias2 = bias1.reshape(1, NO)

    return _tc_matmul_bias(c2, wt, bias2, bm=128)


# speedup vs baseline: 1.7076x; 1.0019x over previous
import functools

import jax
import jax.numpy as jnp
from jax import lax
from jax.experimental import pallas as pl
from jax.experimental.pallas import tpu as pltpu
from jax.experimental.pallas import tpu_sc as plsc


def _sc_gather(table, idx, b_per_w, nc):
    B = idx.shape[0]
    D = table.shape[1]
    mesh = plsc.VectorSubcoreMesh(core_axis_name="c", subcore_axis_name="s")

    @functools.partial(
        pl.kernel,
        mesh=mesh,
        out_type=jax.ShapeDtypeStruct((B, D), jnp.float32),
        scratch_types=[
            pltpu.VMEM((b_per_w,), jnp.int32),
            pltpu.SMEM((b_per_w,), jnp.int32),
            pltpu.VMEM((b_per_w, D), jnp.float32),
            pltpu.SemaphoreType.DMA,
        ],
        compiler_params=pltpu.CompilerParams(use_tc_tiling_on_sc=True),
    )
    def gather_kernel(table_hbm, idx_hbm, out_hbm, idx_v, idx_s, rows_v, sem):
        wid = lax.axis_index("s") * nc + lax.axis_index("c")
        base = wid * b_per_w
        pltpu.sync_copy(idx_hbm.at[pl.ds(base, b_per_w)], idx_v)

        def issue_chunk(j, carry):
            vec = idx_v[pl.ds(j * 16, 16)]
            for lane in range(16):
                g = vec[lane]
                pltpu.make_async_copy(
                    table_hbm.at[pl.ds(g, 1)],
                    rows_v.at[pl.ds(j * 16 + lane, 1)],
                    sem,
                ).start()
            return carry

        lax.fori_loop(0, b_per_w // 16, issue_chunk, 0)
        pltpu.make_async_copy(table_hbm.at[pl.ds(0, b_per_w)], rows_v, sem).wait()
        pltpu.sync_copy(rows_v, out_hbm.at[pl.ds(base, b_per_w)])

    return gather_kernel(table, idx)


def _tc_matmul_bias(c2, wt, bias2, bm):
    M, K = c2.shape
    N = wt.shape[0]

    def body(a_ref, wt_ref, b_ref, o_ref):
        acc = lax.dot_general(
            a_ref[...], wt_ref[...],
            dimension_numbers=(((1,), (1,)), ((), ())),
            preferred_element_type=jnp.float32,
        )
        o_ref[...] = acc + b_ref[...]

    return pl.pallas_call(
        body,
        grid=(M // bm,),
        in_specs=[
            pl.BlockSpec((bm, K), lambda i: (i, 0)),
            pl.BlockSpec((N, K), lambda i: (0, 0)),
            pl.BlockSpec((1, N), lambda i: (0, 0)),
        ],
        out_specs=pl.BlockSpec((bm, N), lambda i: (i, 0)),
        out_shape=jax.ShapeDtypeStruct((M, N), jnp.float32),
        compiler_params=pltpu.CompilerParams(
            dimension_semantics=("parallel",),
        ),
    )(c2, wt, bias2)


def kernel(cellgene_embedding, genes_oi, weight1, bias1):
    M, G, NI = cellgene_embedding.shape
    NO = bias1.shape[0]

    info = plsc.get_sparse_core_info()
    nw = info.num_cores * info.num_subcores
    chunk = 8 * nw
    Bp = ((G + chunk - 1) // chunk) * chunk
    idx = jnp.pad(genes_oi.astype(jnp.int32), (0, Bp - G))

    gathered = _sc_gather(weight1, idx, Bp // nw, info.num_cores)
    wt = gathered[:G].reshape(G * NI, NO).T
    c2 = cellgene_embedding.reshape(M, G * NI)
    bias2 = bias1.reshape(1, NO)

    return _tc_matmul_bias(c2, wt, bias2, bm=128)


# speedup vs baseline: 1.7501x; 1.0249x over previous
import jax
import jax.numpy as jnp
from jax import lax
from jax.experimental import pallas as pl
from jax.experimental.pallas import tpu as pltpu


def _tc_gather(table, idx):
    B = idx.shape[0]
    D = table.shape[1]

    def body(idx_ref, table_ref, out_ref, sem):
        def issue(i, carry):
            g = idx_ref[i]
            pltpu.make_async_copy(
                table_ref.at[pl.ds(g, 1)], out_ref.at[pl.ds(i, 1)], sem
            ).start()
            return carry

        lax.fori_loop(0, B, issue, 0, unroll=8)

        def drain(i, carry):
            pltpu.make_async_copy(
                table_ref.at[pl.ds(0, 1)], out_ref.at[pl.ds(i, 1)], sem
            ).wait()
            return carry

        lax.fori_loop(0, B, drain, 0, unroll=8)

    return pl.pallas_call(
        body,
        in_specs=[
            pl.BlockSpec(memory_space=pltpu.SMEM),
            pl.BlockSpec(memory_space=pl.ANY),
        ],
        out_specs=pl.BlockSpec(memory_space=pltpu.VMEM),
        out_shape=jax.ShapeDtypeStruct((B, D), jnp.float32),
        scratch_shapes=[pltpu.SemaphoreType.DMA],
    )(idx, table)


def _tc_matmul_bias(c2, wt, bias2, bm):
    M, K = c2.shape
    N = wt.shape[0]

    def body(a_ref, wt_ref, b_ref, o_ref):
        acc = lax.dot_general(
            a_ref[...], wt_ref[...],
            dimension_numbers=(((1,), (1,)), ((), ())),
            preferred_element_type=jnp.float32,
        )
        o_ref[...] = acc + b_ref[...]

    return pl.pallas_call(
        body,
        grid=(M // bm,),
        in_specs=[
            pl.BlockSpec((bm, K), lambda i: (i, 0)),
            pl.BlockSpec((N, K), lambda i: (0, 0)),
            pl.BlockSpec((1, N), lambda i: (0, 0)),
        ],
        out_specs=pl.BlockSpec((bm, N), lambda i: (i, 0)),
        out_shape=jax.ShapeDtypeStruct((M, N), jnp.float32),
        compiler_params=pltpu.CompilerParams(
            dimension_semantics=("parallel",),
        ),
    )(c2, wt, bias2)


def kernel(cellgene_embedding, genes_oi, weight1, bias1):
    M, G, NI = cellgene_embedding.shape
    NO = bias1.shape[0]

    idx = genes_oi.astype(jnp.int32)
    gathered = _tc_gather(weight1, idx)
    wt = gathered.reshape(G * NI, NO).T
    c2 = cellgene_embedding.reshape(M, G * NI)
    bias2 = bias1.reshape(1, NO)

    return _tc_matmul_bias(c2, wt, bias2, bm=128)
